# Initial kernel scaffold; baseline (speedup 1.0000x reference)
import jax
import jax.numpy as jnp
from jax.experimental import pallas as pl


def kernel(x, W0, g0, b0, W1, g1, b1, W2, g2, b2, W3, g3, b3, W4, g4, b4, W5, g5, b5, W6, g6, b6, fc1_w, fc1_b, fc2_w, fc2_b):
    raise NotImplementedError("write your pallas kernel here")


# speedup vs baseline: 24.1091x; 24.1091x over previous
import functools

import jax
import jax.numpy as jnp
from jax.experimental import pallas as pl
from jax.experimental.pallas import tpu as pltpu

_B = 32
_N0 = 16384
_EPS = 1e-5
_CHUNK = 2048
_NCHUNK = _N0 // _CHUNK
_HALO = 128


def _offs(n):
    return [o for o in range(1, 5) if o < n]


def _mv(z, axis):
    n = z.shape[axis]
    offs = _offs(n)
    seq = [(o, sgn) for o in offs for sgn in (1, -1)]
    if len(offs) == 4:
        acc = jnp.roll(z, 1, axis=axis)
        for o, sgn in seq[1:]:
            acc = acc + jnp.roll(z, sgn * o, axis=axis)
        return acc * (-0.125)
    w = jnp.float32(-2.0 / (4 * len(offs)))
    acc = w * jnp.roll(z, 1, axis=axis)
    for o, sgn in seq[1:]:
        acc = acc + w * jnp.roll(z, sgn * o, axis=axis)
    return acc


def _cheb_terms(z, axis):
    t0 = z
    t1 = _mv(t0, axis)
    t2 = 2.0 * _mv(t1, axis) - t0
    t3 = 2.0 * _mv(t2, axis) - t1
    return t0, t1, t2, t3


def _bf(x):
    return x.astype(jnp.bfloat16).astype(jnp.float32)


def _sel_mat():
    li = jax.lax.broadcasted_iota(jnp.int32, (128, 32), 0)
    qi = jax.lax.broadcasted_iota(jnp.int32, (128, 32), 1)
    return (li == 4 * qi).astype(jnp.float32)


def _pool_compact(a):
    c, n = a.shape
    m = jnp.maximum(jnp.maximum(a, jnp.roll(a, -1, axis=1)),
                    jnp.maximum(jnp.roll(a, -2, axis=1),
                                jnp.roll(a, -3, axis=1)))
    m2 = m.reshape(c * n // 128, 128)
    return jnp.dot(m2, _sel_mat(), preferred_element_type=jnp.float32,
                 precision=jax.lax.Precision.HIGHEST)


def _accum(vec, bidx, acc_ref, out_ref):
    @pl.when(bidx == 0)
    def _():
        acc_ref[...] = jnp.zeros_like(acc_ref)

    acc_ref[...] = acc_ref[...] + vec
    out_ref[...] = acc_ref[...]


def _recompute_y(x_ref, wt_ref):
    xb = x_ref[0]
    t0, t1, t2, t3 = _cheb_terms(xb, axis=1)
    T = jnp.concatenate([t0, t1, t2, t3], axis=0)
    return jnp.dot(wt_ref[...], T, preferred_element_type=jnp.float32)


def _stats0_body(x_ref, sg_ref):
    x = x_ref[...]
    ts = [_bf(t) for t in _cheb_terms(x, axis=1)]
    rows = []
    for k in range(4):
        ent = [jnp.sum(ts[k], keepdims=True).reshape(1, 1)]
        for l in range(4):
            ent.append(jnp.sum(ts[k] * ts[l], keepdims=True).reshape(1, 1))
        rows.append(jnp.concatenate(ent, axis=1))
    sg_ref[...] = jnp.concatenate(rows, axis=0)


def _apply0_body(xp_ref, sg_ref, w0t_ref, g0_ref, b0_ref, x1_ref):
    cidx = pl.program_id(0)
    s = sg_ref[:, 0:1]
    G = sg_ref[:, 1:5]
    w0t = _bf(w0t_ref[...])
    n_tot = jnp.float32(_B * _N0)
    hp = jax.lax.Precision.HIGHEST
    m = jnp.dot(w0t, s, precision=hp) / n_tot
    e2 = jnp.sum(jnp.dot(w0t, G, precision=hp) * w0t, axis=1,
                 keepdims=True) / n_tot
    v = e2 - m * m
    mb = m[:, 0][None, :, None]
    den = jnp.sqrt(v + _EPS)[:, 0][None, :, None]
    gb = g0_ref[...][:, 0][None, :, None]
    bb = b0_ref[...][:, 0][None, :, None]

    xl = xp_ref[:, pl.ds(cidx * _CHUNK, _CHUNK + 2 * _HALO)]
    ts = _cheb_terms(xl, axis=1)
    y0 = None
    for k in range(4):
        t = _bf(ts[k][:, _HALO:_HALO + _CHUNK])
        c = t[:, None, :] * w0t[:, k][None, :, None]
        y0 = c if y0 is None else y0 + c
    a = jnp.maximum(gb * (y0 - mb) / den + bb, 0.0)
    am = jnp.maximum(jnp.maximum(a, jnp.roll(a, -1, axis=2)),
                     jnp.maximum(jnp.roll(a, -2, axis=2),
                                 jnp.roll(a, -3, axis=2)))
    m2 = am.reshape(_B * 32 * _CHUNK // 128, 128)
    p = jnp.dot(m2, _sel_mat(), preferred_element_type=jnp.float32,
                 precision=jax.lax.Precision.HIGHEST)
    x1_ref[...] = p.reshape(_B * 32, _CHUNK // 128, 32)


def _statsA_body(x_ref, wt_ref, out_ref, acc_ref):
    y = _recompute_y(x_ref, wt_ref)
    _accum(jnp.sum(y, axis=1, keepdims=True), pl.program_id(0), acc_ref,
           out_ref)


def _statsB_body(x_ref, wt_ref, sa_ref, out_ref, acc_ref, *, nin):
    m = sa_ref[...] / jnp.float32(_B * nin)
    y = _recompute_y(x_ref, wt_ref)
    d = y - m
    _accum(jnp.sum(d * d, axis=1, keepdims=True), pl.program_id(0), acc_ref,
           out_ref)


def _apply_body(x_ref, sa_ref, sb_ref, wt_ref, g_ref, b_ref, xo_ref, *, nin):
    cout = wt_ref.shape[0]
    count = jnp.float32(_B * nin)
    m = sa_ref[...] / count
    v = sb_ref[...] / count
    y = _recompute_y(x_ref, wt_ref)
    a = jnp.maximum(g_ref[...] * (y - m) / jnp.sqrt(v + _EPS) + b_ref[...],
                    0.0)
    p = _pool_compact(a)
    xo_ref[0] = p.reshape(cout, nin // 128, 32)


def _tail_layer(h, W, g, b):
    t0, t1, t2, t3 = _cheb_terms(h, axis=1)
    T = jnp.concatenate([t0, t1, t2, t3], axis=-1)
    y = jax.lax.dot_general(T, W, (((2,), (0,)), ((), ())),
                            preferred_element_type=jnp.float32)
    m = jnp.mean(y, axis=(0, 1))
    v = jnp.mean((y - m) ** 2, axis=(0, 1))
    a = jnp.maximum(g * (y - m) / jnp.sqrt(v + _EPS) + b, 0.0)
    bsz, n, c = a.shape
    return jnp.max(a.reshape(bsz, n // 4, 4, c), axis=2)


def _tail_body(x4_ref, w4_ref, g4_ref, b4_ref, w5_ref, g5_ref, b5_ref,
               w6_ref, g6_ref, b6_ref, fc1w_ref, fc1b_ref, fc2w_ref,
               fc2b_ref, out_ref):
    h = x4_ref[...]
    h = _tail_layer(h, w4_ref[...], g4_ref[...], b4_ref[...])
    h = _tail_layer(h, w5_ref[...], g5_ref[...], b5_ref[...])
    h = _tail_layer(h, w6_ref[...], g6_ref[...], b6_ref[...])
    hs = h.reshape(_B, 256)
    f1 = jnp.maximum(jnp.dot(hs, fc1w_ref[...],
                             preferred_element_type=jnp.float32)
                     + fc1b_ref[...], 0.0)
    f2 = jnp.maximum(jnp.dot(f1, fc2w_ref[...],
                             preferred_element_type=jnp.float32)
                     + fc2b_ref[...], 0.0)
    out_ref[...] = f2


def _full(shape):
    return pl.BlockSpec(shape, lambda b: (0,) * len(shape))


def kernel(x, W0, g0, b0, W1, g1, b1, W2, g2, b2, W3, g3, b3, W4, g4, b4,
           W5, g5, b5, W6, g6, b6, fc1_w, fc1_b, fc2_w, fc2_b):
    f32 = jnp.float32
    col = lambda v: v.reshape(-1, 1)

    sg = pl.pallas_call(
        _stats0_body,
        out_shape=jax.ShapeDtypeStruct((4, 5), f32),
    )(x)

    xp = jnp.concatenate([x[:, -_HALO:], x, x[:, :_HALO]], axis=1)
    x1r = pl.pallas_call(
        _apply0_body,
        grid=(_NCHUNK,),
        in_specs=[
            _full((_B, _N0 + 2 * _HALO)),
            _full((4, 5)), _full((32, 4)), _full((32, 1)), _full((32, 1)),
        ],
        out_specs=pl.BlockSpec((_B * 32, _CHUNK // 128, 32),
                               lambda c: (0, c, 0)),
        out_shape=jax.ShapeDtypeStruct((_B * 32, _N0 // 128, 32), f32),
    )(xp, sg, W0.T, col(g0), col(b0))
    x1 = x1r.reshape(_B, 32, 4096)

    def stats_call(xi, wt, cin, nin, cout):
        xspec = pl.BlockSpec((1, cin, nin), lambda b: (b, 0, 0))
        sa = pl.pallas_call(
            _statsA_body,
            grid=(_B,),
            in_specs=[xspec, _full(wt.shape)],
            out_specs=_full((cout, 1)),
            out_shape=jax.ShapeDtypeStruct((cout, 1), f32),
            scratch_shapes=[pltpu.VMEM((cout, 1), f32)],
        )(xi, wt)
        sb = pl.pallas_call(
            functools.partial(_statsB_body, nin=nin),
            grid=(_B,),
            in_specs=[xspec, _full(wt.shape), _full((cout, 1))],
            out_specs=_full((cout, 1)),
            out_shape=jax.ShapeDtypeStruct((cout, 1), f32),
            scratch_shapes=[pltpu.VMEM((cout, 1), f32)],
        )(xi, wt, sa)
        return sa, sb

    def apply_call(xi, stats, wt, g, b, cin, nin, cout):
        body = functools.partial(_apply_body, nin=nin)
        xo = pl.pallas_call(
            body,
            grid=(_B,),
            in_specs=[
                pl.BlockSpec((1, cin, nin), lambda b: (b, 0, 0)),
                _full((cout, 1)), _full((cout, 1)), _full(wt.shape),
                _full((cout, 1)), _full((cout, 1)),
            ],
            out_specs=pl.BlockSpec((1, cout, nin // 128, 32),
                                   lambda b: (b, 0, 0, 0)),
            out_shape=jax.ShapeDtypeStruct((_B, cout, nin // 128, 32), f32),
        )(xi, *stats, wt, g, b)
        return xo.reshape(_B, cout, nin // 4)

    stats1 = stats_call(x1, W1.T, 32, 4096, 64)
    x2 = apply_call(x1, stats1, W1.T, col(g1), col(b1), 32, 4096, 64)

    stats2 = stats_call(x2, W2.T, 64, 1024, 128)
    x3 = apply_call(x2, stats2, W2.T, col(g2), col(b2), 64, 1024, 128)

    stats3 = stats_call(x3, W3.T, 128, 256, 256)
    x4c = apply_call(x3, stats3, W3.T, col(g3), col(b3), 128, 256, 256)

    x4 = x4c.transpose(0, 2, 1)

    out = pl.pallas_call(
        _tail_body,
        out_shape=jax.ShapeDtypeStruct((_B, 64), f32),
    )(x4, W4, g4, b4, W5, g5, b5, W6, g6, b6, fc1_w, fc1_b, fc2_w, fc2_b)
    return out
